# baseline (speedup 1.0000x reference)
import jax
import jax.numpy as jnp
from jax import lax
from jax.experimental import pallas as pl
from jax.experimental.pallas import tpu as pltpu

_N = 10000
_ROWS = 8
_COLS = 1280

_ROTATIONS = ((13, 15, 26, 6), (17, 29, 16, 24))
_KEY_LO = 42


def _rotl(v, r):
    return lax.shift_left(v, jnp.uint32(r)) | lax.shift_right_logical(
        v, jnp.uint32(32 - r))


def _keep_mask(x1):
    k0 = jnp.uint32(0)
    k1 = jnp.uint32(_KEY_LO)
    ks = (k0, k1, k0 ^ k1 ^ jnp.uint32(0x1BD11BDA))
    x0 = jnp.zeros(x1.shape, jnp.uint32) + ks[0]
    x1 = x1 + ks[1]
    for i in range(5):
        for r in _ROTATIONS[i % 2]:
            x0 = x0 + x1
            x1 = _rotl(x1, r)
            x1 = x1 ^ x0
        x0 = x0 + ks[(i + 1) % 3]
        x1 = x1 + ks[(i + 2) % 3] + jnp.uint32(i + 1)
    bits = x0 ^ x1
    return lax.shift_right_logical(bits, jnp.uint32(9)) > jnp.uint32(419430)


_XCHUNKS = 4
_ECHUNKS = 2


def _body(x_ref, e_ref, y_ref, xo_ref, eo_ref, yo_ref, m_ref, *sems):
    copies = []
    xc = x_ref.shape[0] // _XCHUNKS
    for c in range(_XCHUNKS):
        copies.append(pltpu.make_async_copy(
            x_ref.at[pl.ds(c * xc, xc)], xo_ref.at[pl.ds(c * xc, xc)],
            sems[c]))
    ec = e_ref.shape[1] // _ECHUNKS
    for c in range(_ECHUNKS):
        copies.append(pltpu.make_async_copy(
            e_ref.at[:, pl.ds(c * ec, ec)], eo_ref.at[:, pl.ds(c * ec, ec)],
            sems[_XCHUNKS + c]))
    copies.append(pltpu.make_async_copy(y_ref, yo_ref,
                                        sems[_XCHUNKS + _ECHUNKS]))
    for c in copies:
        c.start()
    cnt = (lax.broadcasted_iota(jnp.uint32, (_ROWS, _COLS), 0) * _COLS
           + lax.broadcasted_iota(jnp.uint32, (_ROWS, _COLS), 1))
    keep = _keep_mask(cnt)
    for r in range(_ROWS):
        row = jnp.reshape(keep[r:r + 1, :], (_COLS,))
        base = r * _COLS
        if base + _COLS <= _N:
            m_ref[pl.ds(base, _COLS)] = row
        else:
            tail = _N - base
            m_ref[pl.ds(base, tail)] = lax.slice(row, (0,), (tail,))
    for c in copies:
        c.wait()


def kernel(x, y, edge_index):
    x_out, e_out, y_out, m = pl.pallas_call(
        _body,
        in_specs=[
            pl.BlockSpec(memory_space=pltpu.MemorySpace.VMEM),
            pl.BlockSpec(memory_space=pltpu.MemorySpace.VMEM),
            pl.BlockSpec(memory_space=pltpu.MemorySpace.VMEM),
        ],
        out_specs=[
            pl.BlockSpec(memory_space=pltpu.MemorySpace.HBM),
            pl.BlockSpec(memory_space=pltpu.MemorySpace.HBM),
            pl.BlockSpec(memory_space=pltpu.MemorySpace.HBM),
            pl.BlockSpec(memory_space=pltpu.MemorySpace.VMEM),
        ],
        out_shape=[
            jax.ShapeDtypeStruct(x.shape, x.dtype),
            jax.ShapeDtypeStruct(edge_index.shape, edge_index.dtype),
            jax.ShapeDtypeStruct(y.shape, y.dtype),
            jax.ShapeDtypeStruct((_N,), jnp.bool_),
        ],
        scratch_shapes=[pltpu.SemaphoreType.DMA] * (_XCHUNKS + _ECHUNKS + 1),
    )(x, edge_index, y)
    return (x_out, e_out, y_out, m, m)


# speedup vs baseline: 1.0871x; 1.0871x over previous
import jax
import jax.numpy as jnp
from jax import lax
from jax.experimental import pallas as pl
from jax.experimental.pallas import tpu as pltpu

_N = 10000
_ROWS = 8
_COLS = 1280

_XCHUNKS = 8
_ECHUNKS = 4
_NCHUNKS = _XCHUNKS + _ECHUNKS + 1

_ROTATIONS = ((13, 15, 26, 6), (17, 29, 16, 24))
_KEY_LO = 42


def _rotl(v, r):
    return lax.shift_left(v, jnp.uint32(r)) | lax.shift_right_logical(
        v, jnp.uint32(32 - r))


def _keep_mask(x1):
    k0 = jnp.uint32(0)
    k1 = jnp.uint32(_KEY_LO)
    ks = (k0, k1, k0 ^ k1 ^ jnp.uint32(0x1BD11BDA))
    x0 = jnp.zeros(x1.shape, jnp.uint32) + ks[0]
    x1 = x1 + ks[1]
    for i in range(5):
        for r in _ROTATIONS[i % 2]:
            x0 = x0 + x1
            x1 = _rotl(x1, r)
            x1 = x1 ^ x0
        x0 = x0 + ks[(i + 1) % 3]
        x1 = x1 + ks[(i + 2) % 3] + jnp.uint32(i + 1)
    bits = x0 ^ x1
    return lax.shift_right_logical(bits, jnp.uint32(9)) > jnp.uint32(419430)


def _body(x_ref, e_ref, y_ref, xo_ref, eo_ref, yo_ref, m_ref,
          xbuf, ebuf, ybuf, rsem, wsem):
    xc = x_ref.shape[0] // _XCHUNKS
    ec = e_ref.shape[1] // _ECHUNKS
    reads, writes = [], []
    for c in range(_XCHUNKS):
        sl = pl.ds(c * xc, xc)
        reads.append(pltpu.make_async_copy(
            x_ref.at[sl], xbuf.at[sl], rsem.at[c]))
        writes.append(pltpu.make_async_copy(
            xbuf.at[sl], xo_ref.at[sl], wsem.at[c]))
    for c in range(_ECHUNKS):
        sl = pl.ds(c * ec, ec)
        reads.append(pltpu.make_async_copy(
            e_ref.at[:, sl], ebuf.at[:, sl], rsem.at[_XCHUNKS + c]))
        writes.append(pltpu.make_async_copy(
            ebuf.at[:, sl], eo_ref.at[:, sl], wsem.at[_XCHUNKS + c]))
    reads.append(pltpu.make_async_copy(y_ref, ybuf, rsem.at[_NCHUNKS - 1]))
    writes.append(pltpu.make_async_copy(ybuf, yo_ref, wsem.at[_NCHUNKS - 1]))

    for r in reads:
        r.start()

    cnt = (lax.broadcasted_iota(jnp.uint32, (_ROWS, _COLS), 0) * _COLS
           + lax.broadcasted_iota(jnp.uint32, (_ROWS, _COLS), 1))
    keep = _keep_mask(cnt)
    for r in range(_ROWS):
        row = jnp.reshape(keep[r:r + 1, :], (_COLS,))
        base = r * _COLS
        if base + _COLS <= _N:
            m_ref[pl.ds(base, _COLS)] = row
        else:
            tail = _N - base
            m_ref[pl.ds(base, tail)] = lax.slice(row, (0,), (tail,))

    for c in range(_NCHUNKS):
        reads[c].wait()
        writes[c].start()
    for c in range(_NCHUNKS):
        writes[c].wait()


def kernel(x, y, edge_index):
    x_out, e_out, y_out, m = pl.pallas_call(
        _body,
        in_specs=[
            pl.BlockSpec(memory_space=pltpu.MemorySpace.HBM),
            pl.BlockSpec(memory_space=pltpu.MemorySpace.HBM),
            pl.BlockSpec(memory_space=pltpu.MemorySpace.HBM),
        ],
        out_specs=[
            pl.BlockSpec(memory_space=pltpu.MemorySpace.HBM),
            pl.BlockSpec(memory_space=pltpu.MemorySpace.HBM),
            pl.BlockSpec(memory_space=pltpu.MemorySpace.HBM),
            pl.BlockSpec(memory_space=pltpu.MemorySpace.VMEM),
        ],
        out_shape=[
            jax.ShapeDtypeStruct(x.shape, x.dtype),
            jax.ShapeDtypeStruct(edge_index.shape, edge_index.dtype),
            jax.ShapeDtypeStruct(y.shape, y.dtype),
            jax.ShapeDtypeStruct((_N,), jnp.bool_),
        ],
        scratch_shapes=[
            pltpu.VMEM(x.shape, x.dtype),
            pltpu.VMEM(edge_index.shape, edge_index.dtype),
            pltpu.VMEM(y.shape, y.dtype),
            pltpu.SemaphoreType.DMA((_NCHUNKS,)),
            pltpu.SemaphoreType.DMA((_NCHUNKS,)),
        ],
    )(x, edge_index, y)
    return (x_out, e_out, y_out, m, m)
